# Initial kernel scaffold; baseline (speedup 1.0000x reference)
import jax
import jax.numpy as jnp
from jax.experimental import pallas as pl


def kernel(inputs, W):
    raise NotImplementedError("write your pallas kernel here")


# speedup vs baseline: 1.1779x; 1.1779x over previous
import functools

import jax
import jax.numpy as jnp
from jax import lax
from jax.experimental import pallas as pl
from jax.experimental.pallas import tpu as pltpu
from jax.experimental.pallas import tpu_sc as plsc

_V = 8192
_D = 256
_B = 16
_S = 576
_N = _B * _S
_CHUNK = 1024
_NCHUNK = _V // _CHUNK

_NC, _NS = 2, 16
_NW = _NC * _NS
_BPW = _N // _NW
_GCH = 96
_NGC = _BPW // _GCH


def _wnorm_body(w_ref, out_ref):
    w = w_ref[...]
    n = jnp.sqrt(jnp.sum(w * w, axis=1, keepdims=True))
    out_ref[...] = w / jnp.maximum(n, 1e-12)


def _normalize_w(W):
    return pl.pallas_call(
        _wnorm_body,
        out_shape=jax.ShapeDtypeStruct((_V, _D), jnp.float32),
    )(W)


_WINS = (0, 2736, 5472, 8192)


def _rne_bf16_f32(a):
    u = lax.bitcast_convert_type(a, jnp.uint32)
    lsb = (u >> 16) & jnp.uint32(1)
    r = (u + jnp.uint32(0x7FFF) + lsb) & jnp.uint32(0xFFFF0000)
    return lax.bitcast_convert_type(r, jnp.float32)


def _rne_bf16(a):
    return _rne_bf16_f32(a).astype(jnp.bfloat16)


def _argmin_body(x_ref, w_ref, idx_ref):
    x = x_ref[0]
    xn = jnp.sqrt(jnp.sum(x * x, axis=0, keepdims=True))
    xh = x / jnp.maximum(xn, 1e-12)
    x16 = _rne_bf16(xh)
    bv = jnp.full((1, _S), jnp.inf, jnp.float32)
    bi = jnp.zeros((1, _S), jnp.int32)
    dims = (((1,), (0,)), ((), ()))
    for k in range(len(_WINS) - 1):
        start = _WINS[k]
        n = _WINS[k + 1] - start
        w = w_ref[pl.ds(start, n), :]
        cs = lax.dot_general(_rne_bf16(w), x16, dims,
                             preferred_element_type=jnp.float32)
        d = 1.0 - cs
        lmin = jnp.min(d, axis=0, keepdims=True)
        rows = lax.broadcasted_iota(jnp.int32, d.shape, 0) + start
        li = jnp.min(jnp.where(d == lmin, rows, jnp.int32(2**30)),
                     axis=0, keepdims=True)
        take = lmin < bv
        bv = jnp.where(take, lmin, bv)
        bv = _rne_bf16_f32(bv)
        bi = jnp.where(take, li, bi)
    idx_ref[0] = bi


def _argmin(inputs, W_hat):
    return pl.pallas_call(
        _argmin_body,
        grid=(_B,),
        in_specs=[
            pl.BlockSpec((1, _D, _S), lambda b: (b, 0, 0)),
            pl.BlockSpec((_V, _D), lambda b: (0, 0)),
        ],
        out_specs=pl.BlockSpec((1, 1, _S), lambda b: (b, 0, 0)),
        out_shape=jax.ShapeDtypeStruct((_B, 1, _S), jnp.int32),
        compiler_params=pltpu.CompilerParams(
            dimension_semantics=("arbitrary",)),
    )(inputs, W_hat)


def _sc_gather_kernel():
    mesh = plsc.VectorSubcoreMesh(core_axis_name="c", subcore_axis_name="s")

    @functools.partial(
        pl.kernel,
        mesh=mesh,
        out_type=(
            jax.ShapeDtypeStruct((_N, _D), jnp.float32),
            jax.ShapeDtypeStruct((_NC, _V), jnp.float32),
        ),
        scratch_types=(
            [pltpu.VMEM((_GCH,), jnp.int32) for _ in range(_NGC)]
            + [pltpu.VMEM((_GCH, _D), jnp.float32) for _ in range(_NGC)]
            + [pltpu.VMEM((_GCH,), jnp.float32)]
            + [pltpu.VMEM((_V,), jnp.float32)]
            + [pltpu.VMEM_SHARED((_V,), jnp.float32)]
            + [pltpu.SemaphoreType.DMA for _ in range(_NGC)]
        ),
    )
    def k(idx_hbm, w_hbm, q_hbm, part_hbm,
          i0, i1, i2, r0, r1, r2, ones_u, zbuf, shared, s0, s1, s2):
        cid = lax.axis_index("c")
        sid = lax.axis_index("s")
        wid = sid * _NC + cid
        base = wid * _BPW
        ibufs = (i0, i1, i2)
        rbufs = (r0, r1, r2)
        sems = (s0, s1, s2)
        for c in range(_NGC):
            pltpu.sync_copy(idx_hbm.at[pl.ds(base + c * _GCH, _GCH)],
                            ibufs[c])
        copies = [pltpu.async_copy(w_hbm.at[ibufs[c]], rbufs[c], sems[c])
                  for c in range(_NGC)]

        ones16 = jnp.ones((16,), jnp.float32)
        zeros16 = jnp.zeros((16,), jnp.float32)

        def _ones(i, carry):
            ones_u[pl.ds(i * 16, 16)] = ones16
            return carry

        lax.fori_loop(0, _GCH // 16, _ones, 0)

        @pl.when(sid == 0)
        def _():
            def _zero(i, carry):
                zbuf[pl.ds(i * 16, 16)] = zeros16
                return carry

            lax.fori_loop(0, _V // 16, _zero, 0)
            pltpu.sync_copy(zbuf, shared)

        plsc.subcore_barrier()
        for c in range(_NGC):
            pltpu.sync_copy(ones_u, shared.at[ibufs[c]], add=True)
        plsc.subcore_barrier()

        @pl.when(sid == 0)
        def _():
            pltpu.sync_copy(shared, part_hbm.at[cid])

        for c in range(_NGC):
            copies[c].wait()
            pltpu.sync_copy(rbufs[c], q_hbm.at[pl.ds(base + c * _GCH, _GCH)])

    return k


def _fin_body(q_ref, x_ref, part_ref, out_ref, loss_ref, perp_ref, acc_ref):
    b = pl.program_id(0)
    qt = q_ref[0].T
    x = x_ref[0]
    diff = qt - x
    out_ref[0] = x + diff
    s = jnp.sum(diff * diff)

    @pl.when(b == 0)
    def _():
        acc_ref[0, 0] = s

    @pl.when(b > 0)
    def _():
        acc_ref[0, 0] = acc_ref[0, 0] + s

    @pl.when(b == _B - 1)
    def _():
        total = acc_ref[0, 0]
        m = total / jnp.float32(_N * _D)
        loss_ref[...] = jnp.reshape(m + jnp.float32(0.25) * m, (1, 1))
        counts = jnp.sum(part_ref[...], axis=0)
        p = counts / jnp.float32(_N)
        ent = jnp.sum(p * jnp.log(p + 1e-10))
        perp_ref[...] = jnp.reshape(jnp.exp(-ent), (1, 1))


def _finalize(q3, inputs, part):
    return pl.pallas_call(
        _fin_body,
        grid=(_B,),
        in_specs=[
            pl.BlockSpec((1, _S, _D), lambda b: (b, 0, 0)),
            pl.BlockSpec((1, _D, _S), lambda b: (b, 0, 0)),
            pl.BlockSpec((_NC, _V), lambda b: (0, 0)),
        ],
        out_specs=[
            pl.BlockSpec((1, _D, _S), lambda b: (b, 0, 0)),
            pl.BlockSpec((1, 1), lambda b: (0, 0)),
            pl.BlockSpec((1, 1), lambda b: (0, 0)),
        ],
        out_shape=[
            jax.ShapeDtypeStruct((_B, _D, _S), jnp.float32),
            jax.ShapeDtypeStruct((1, 1), jnp.float32),
            jax.ShapeDtypeStruct((1, 1), jnp.float32),
        ],
        scratch_shapes=[pltpu.SMEM((1, 1), jnp.float32)],
        compiler_params=pltpu.CompilerParams(
            dimension_semantics=("arbitrary",)),
    )(q3, inputs, part)


def kernel(inputs, W):
    W_hat = _normalize_w(W)
    idx3 = _argmin(inputs, W_hat)
    idx_flat = idx3.reshape(_N)
    q_flat, part = _sc_gather_kernel()(idx_flat, W)
    q3 = q_flat.reshape(_B, _S, _D)
    quant, loss, perp = _finalize(q3, inputs, part)
    return quant, loss[0, 0], perp[0, 0]


# speedup vs baseline: 1.2953x; 1.0996x over previous
import functools

import jax
import jax.numpy as jnp
from jax import lax
from jax.experimental import pallas as pl
from jax.experimental.pallas import tpu as pltpu
from jax.experimental.pallas import tpu_sc as plsc

_V = 8192
_D = 256
_B = 16
_S = 576
_N = _B * _S
_CHUNK = 1024
_NCHUNK = _V // _CHUNK

_NC, _NS = 2, 16
_NW = _NC * _NS
_BPW = _N // _NW
_GCH = 96
_NGC = _BPW // _GCH


def _wnorm_body(w_ref, out_ref):
    w = w_ref[...]
    n = jnp.sqrt(jnp.sum(w * w, axis=1, keepdims=True))
    out_ref[...] = _rne_bf16(w / jnp.maximum(n, 1e-12))


def _normalize_w(W):
    return pl.pallas_call(
        _wnorm_body,
        out_shape=jax.ShapeDtypeStruct((_V, _D), jnp.bfloat16),
    )(W)


_WINS = (0, 2736, 5472, 8192)


def _rne_bf16_f32(a):
    u = lax.bitcast_convert_type(a, jnp.uint32)
    lsb = (u >> 16) & jnp.uint32(1)
    r = (u + jnp.uint32(0x7FFF) + lsb) & jnp.uint32(0xFFFF0000)
    return lax.bitcast_convert_type(r, jnp.float32)


def _rne_bf16(a):
    return _rne_bf16_f32(a).astype(jnp.bfloat16)


def _argmin_body(x_ref, w_ref, idx_ref):
    x = x_ref[0]
    xn = jnp.sqrt(jnp.sum(x * x, axis=0, keepdims=True))
    xh = x / jnp.maximum(xn, 1e-12)
    x16 = _rne_bf16(xh)
    bv = jnp.full((1, _S), jnp.inf, jnp.float32)
    bi = jnp.zeros((1, _S), jnp.int32)
    dims = (((1,), (0,)), ((), ()))
    for k in range(len(_WINS) - 1):
        start = _WINS[k]
        n = _WINS[k + 1] - start
        w = w_ref[pl.ds(start, n), :]
        cs = lax.dot_general(w, x16, dims,
                             preferred_element_type=jnp.float32)
        d = 1.0 - cs
        lmin = jnp.min(d, axis=0, keepdims=True)
        rows = lax.broadcasted_iota(jnp.int32, d.shape, 0) + start
        li = jnp.min(jnp.where(d == lmin, rows, jnp.int32(2**30)),
                     axis=0, keepdims=True)
        take = lmin < bv
        bv = jnp.where(take, lmin, bv)
        bv = _rne_bf16_f32(bv)
        bi = jnp.where(take, li, bi)
    idx_ref[0] = bi


def _argmin(inputs, W_hat):
    return pl.pallas_call(
        _argmin_body,
        grid=(_B,),
        in_specs=[
            pl.BlockSpec((1, _D, _S), lambda b: (b, 0, 0)),
            pl.BlockSpec((_V, _D), lambda b: (0, 0)),
        ],
        out_specs=pl.BlockSpec((1, 1, _S), lambda b: (b, 0, 0)),
        out_shape=jax.ShapeDtypeStruct((_B, 1, _S), jnp.int32),
        compiler_params=pltpu.CompilerParams(
            dimension_semantics=("arbitrary",)),
    )(inputs, W_hat)


def _sc_gather_kernel():
    mesh = plsc.VectorSubcoreMesh(core_axis_name="c", subcore_axis_name="s")

    @functools.partial(
        pl.kernel,
        mesh=mesh,
        out_type=(
            jax.ShapeDtypeStruct((_N, _D), jnp.float32),
            jax.ShapeDtypeStruct((_NC, _V), jnp.float32),
        ),
        scratch_types=(
            [pltpu.VMEM((_GCH,), jnp.int32) for _ in range(_NGC)]
            + [pltpu.VMEM((_GCH, _D), jnp.float32) for _ in range(_NGC)]
            + [pltpu.VMEM((_GCH,), jnp.float32)]
            + [pltpu.VMEM((_V,), jnp.float32)]
            + [pltpu.VMEM_SHARED((_V,), jnp.float32)]
            + [pltpu.SemaphoreType.DMA for _ in range(_NGC)]
        ),
    )
    def k(idx_hbm, w_hbm, q_hbm, part_hbm,
          i0, i1, i2, r0, r1, r2, ones_u, zbuf, shared, s0, s1, s2):
        cid = lax.axis_index("c")
        sid = lax.axis_index("s")
        wid = sid * _NC + cid
        base = wid * _BPW
        ibufs = (i0, i1, i2)
        rbufs = (r0, r1, r2)
        sems = (s0, s1, s2)
        for c in range(_NGC):
            pltpu.sync_copy(idx_hbm.at[pl.ds(base + c * _GCH, _GCH)],
                            ibufs[c])
        copies = [pltpu.async_copy(w_hbm.at[ibufs[c]], rbufs[c], sems[c])
                  for c in range(_NGC)]

        ones16 = jnp.ones((16,), jnp.float32)
        zeros16 = jnp.zeros((16,), jnp.float32)

        def _ones(i, carry):
            ones_u[pl.ds(i * 16, 16)] = ones16
            return carry

        lax.fori_loop(0, _GCH // 16, _ones, 0)

        @pl.when(sid == 0)
        def _():
            def _zero(i, carry):
                zbuf[pl.ds(i * 16, 16)] = zeros16
                return carry

            lax.fori_loop(0, _V // 16, _zero, 0)
            pltpu.sync_copy(zbuf, shared)

        plsc.subcore_barrier()
        for c in range(_NGC):
            pltpu.sync_copy(ones_u, shared.at[ibufs[c]], add=True)
        plsc.subcore_barrier()

        @pl.when(sid == 0)
        def _():
            pltpu.sync_copy(shared, part_hbm.at[cid])

        for c in range(_NGC):
            copies[c].wait()
            pltpu.sync_copy(rbufs[c], q_hbm.at[pl.ds(base + c * _GCH, _GCH)])

    return k


def _fin_body(q_ref, x_ref, part_ref, out_ref, loss_ref, perp_ref, acc_ref):
    b = pl.program_id(0)
    qt = q_ref[0].T
    x = x_ref[0]
    diff = qt - x
    out_ref[0] = x + diff
    s = jnp.sum(diff * diff)

    @pl.when(b == 0)
    def _():
        acc_ref[0, 0] = s

    @pl.when(b > 0)
    def _():
        acc_ref[0, 0] = acc_ref[0, 0] + s

    @pl.when(b == _B - 1)
    def _():
        total = acc_ref[0, 0]
        m = total / jnp.float32(_N * _D)
        loss_ref[...] = jnp.reshape(m + jnp.float32(0.25) * m, (1, 1))
        counts = jnp.sum(part_ref[...], axis=0)
        p = counts / jnp.float32(_N)
        ent = jnp.sum(p * jnp.log(p + 1e-10))
        perp_ref[...] = jnp.reshape(jnp.exp(-ent), (1, 1))


def _finalize(q3, inputs, part):
    return pl.pallas_call(
        _fin_body,
        grid=(_B,),
        in_specs=[
            pl.BlockSpec((1, _S, _D), lambda b: (b, 0, 0)),
            pl.BlockSpec((1, _D, _S), lambda b: (b, 0, 0)),
            pl.BlockSpec((_NC, _V), lambda b: (0, 0)),
        ],
        out_specs=[
            pl.BlockSpec((1, _D, _S), lambda b: (b, 0, 0)),
            pl.BlockSpec((1, 1), lambda b: (0, 0)),
            pl.BlockSpec((1, 1), lambda b: (0, 0)),
        ],
        out_shape=[
            jax.ShapeDtypeStruct((_B, _D, _S), jnp.float32),
            jax.ShapeDtypeStruct((1, 1), jnp.float32),
            jax.ShapeDtypeStruct((1, 1), jnp.float32),
        ],
        scratch_shapes=[pltpu.SMEM((1, 1), jnp.float32)],
        compiler_params=pltpu.CompilerParams(
            dimension_semantics=("arbitrary",)),
    )(q3, inputs, part)


def kernel(inputs, W):
    W_hat = _normalize_w(W)
    idx3 = _argmin(inputs, W_hat)
    idx_flat = idx3.reshape(_N)
    q_flat, part = _sc_gather_kernel()(idx_flat, W)
    q3 = q_flat.reshape(_B, _S, _D)
    quant, loss, perp = _finalize(q3, inputs, part)
    return quant, loss[0, 0], perp[0, 0]
